# baseline (speedup 1.0000x reference)
import jax
import jax.numpy as jnp
import numpy as np
from jax import lax
from jax.experimental import pallas as pl
from jax.experimental.pallas import tpu as pltpu
from jax.experimental.pallas import tpu_sc as plsc

N = 10000
E = 320000
D = 128

NC = 2
NS = 16
NW = NC * NS
EPW = E // NW
C = 40
NCHUNK = EPW // C
NPAD = 10240
ROWS_PER_TILE = NPAD // NS


def _pack_bf16_words(acc):
    bits = lax.bitcast_convert_type(acc, jnp.uint32)
    bf = (bits + 0x7FFF + ((bits >> 16) & 1)) >> 16
    words = [bf[:, 32 * j:32 * j + 16] | (bf[:, 32 * j + 16:32 * j + 32] << 16)
             for j in range(acc.shape[1] // 32)]
    return lax.bitcast_convert_type(jnp.concatenate(words, axis=1), jnp.int32)


def _pe_body(x_ref, w_ref, b_ref, o0_ref, o1_ref):
    acc = jnp.dot(x_ref[...], w_ref[...], preferred_element_type=jnp.float32)
    acc = acc + b_ref[...]
    o0_ref[...] = _pack_bf16_words(acc[:, :D])
    o1_ref[...] = _pack_bf16_words(acc[:, D:])


def _tc_pe(edge_attr, we0, be0, we1, be1, block_r=2000):
    r, k = edge_attr.shape
    w = jnp.concatenate([we0, we1], axis=1)
    b = jnp.concatenate([be0, be1]).reshape(1, 2 * D)
    return pl.pallas_call(
        _pe_body,
        grid=(r // block_r,),
        in_specs=[
            pl.BlockSpec((block_r, k), lambda i: (i, 0)),
            pl.BlockSpec((k, 2 * D), lambda i: (0, 0)),
            pl.BlockSpec((1, 2 * D), lambda i: (0, 0)),
        ],
        out_specs=[
            pl.BlockSpec((block_r, D // 2), lambda i: (i, 0)),
            pl.BlockSpec((block_r, D // 2), lambda i: (i, 0)),
        ],
        out_shape=[
            jax.ShapeDtypeStruct((r, D // 2), jnp.int32),
            jax.ShapeDtypeStruct((r, D // 2), jnp.int32),
        ],
    )(edge_attr, w, b)


def _proj_body(h_ref, w_ref, o0_ref, o1_ref):
    acc = jnp.dot(h_ref[...], w_ref[...], preferred_element_type=jnp.float32)
    o0_ref[...] = acc[:, :D]
    o1_ref[...] = acc[:, D:]


def _tc_proj(h, wsrc_dst, block_r=400):
    r = h.shape[0]
    return pl.pallas_call(
        _proj_body,
        grid=(r // block_r,),
        in_specs=[
            pl.BlockSpec((block_r, D), lambda i: (i, 0)),
            pl.BlockSpec((D, 2 * D), lambda i: (0, 0)),
        ],
        out_specs=[
            pl.BlockSpec((block_r, D), lambda i: (i, 0)),
            pl.BlockSpec((block_r, D), lambda i: (i, 0)),
        ],
        out_shape=[
            jax.ShapeDtypeStruct((r, D), jnp.float32),
            jax.ShapeDtypeStruct((r, D), jnp.float32),
        ],
    )(h, wsrc_dst)


def _tc_update(h, agg, wu, bu, act, extra_w=None, block_r=400):
    r = h.shape[0]
    nout = 1 if extra_w is None else 3
    out_shape = [jax.ShapeDtypeStruct((r, D), jnp.float32)] * nout
    out_specs = [pl.BlockSpec((block_r, D), lambda i: (i, 0))] * nout

    def body(h_ref, agg_ref, wh_ref, wa_ref, b_ref, ew_ref, *o_refs):
        acc = jnp.dot(h_ref[...], wh_ref[...], preferred_element_type=jnp.float32)
        agg2 = agg_ref[0] + agg_ref[1]
        acc = acc + jnp.dot(agg2, wa_ref[...], preferred_element_type=jnp.float32)
        acc = acc + b_ref[...]
        if act:
            acc = jnp.maximum(acc, 0.0)
        o_refs[0][...] = acc
        if nout == 3:
            pr = jnp.dot(acc, ew_ref[...], preferred_element_type=jnp.float32)
            o_refs[1][...] = pr[:, :D]
            o_refs[2][...] = pr[:, D:]

    ew = extra_w if extra_w is not None else jnp.zeros((D, 2 * D), jnp.float32)
    return pl.pallas_call(
        body,
        grid=(r // block_r,),
        in_specs=[
            pl.BlockSpec((block_r, D), lambda i: (i, 0)),
            pl.BlockSpec((NC, block_r, D), lambda i: (0, i, 0)),
            pl.BlockSpec((D, D), lambda i: (0, 0)),
            pl.BlockSpec((D, D), lambda i: (0, 0)),
            pl.BlockSpec((1, D), lambda i: (0, 0)),
            pl.BlockSpec((D, 2 * D), lambda i: (0, 0)),
        ],
        out_specs=out_specs,
        out_shape=out_shape,
    )(h, agg, wu[:D], wu[D:], bu.reshape(1, D), ew)


def _edge_body(psrc, pdst, pe, src3, dst3, zeros_hbm, out_hbm,
               agg_sh, is0, is1, id0, id1, id2, id3,
               a0, b0, c0, a1, b1, c1,
               sem_i, sem_in0, sem_in1, sem_sc):
    cc = lax.axis_index("c")
    ss = lax.axis_index("s")
    wid = cc * NS + ss
    row0 = ss * ROWS_PER_TILE

    IS = (is0, is1)
    ID = (id0, id1, id2, id3)
    A = (a0, a1)
    B = (b0, b1)
    CB = (c0, c1)
    SIN = (sem_in0, sem_in1)

    def issue_idx(k, bs, bd):
        pltpu.async_copy(src3.at[wid, k], IS[bs], sem_i)
        pltpu.async_copy(dst3.at[wid, k], ID[bd], sem_i)

    def wait_idx(bs, bd):
        pltpu.make_async_copy(src3.at[0, 0], IS[bs], sem_i).wait()
        pltpu.make_async_copy(dst3.at[0, 0], ID[bd], sem_i).wait()

    def issue_in(k, b, bd):
        base = wid * EPW + k * C
        pltpu.async_copy(psrc.at[IS[b]], A[b], SIN[b])
        pltpu.async_copy(pdst.at[ID[bd]], B[b], SIN[b])
        pltpu.async_copy(pe.at[pl.ds(base, C)], CB[b], SIN[b])

    def wait_in(b):
        pltpu.make_async_copy(zeros_hbm.at[pl.ds(0, C)], A[b], SIN[b]).wait()
        pltpu.make_async_copy(zeros_hbm.at[pl.ds(0, C)], B[b], SIN[b]).wait()
        pltpu.make_async_copy(pe.at[pl.ds(0, C)], CB[b], SIN[b]).wait()

    def issue_sc(b, bd):
        pltpu.async_copy(A[b], agg_sh.at[ID[bd]], sem_sc, add=True)

    def wait_sc(b):
        pltpu.make_async_copy(zeros_hbm.at[pl.ds(0, C)], A[b], sem_sc).wait()

    def compute(b):
        ab, bb, cb = A[b], B[b], CB[b]

        def edge(e, _):
            for j in range(D // 32):
                ci = cb[e, pl.ds(16 * j, 16)]
                c32 = plsc.bitcast(ci, jnp.bfloat16)
                clo, chi = plsc.unpack(c32, format=plsc.PackFormat.INTERLEAVED)
                sl_lo = pl.ds(32 * j, 16)
                sl_hi = pl.ds(32 * j + 16, 16)
                vlo = ab[e, sl_lo] + bb[e, sl_lo] + clo
                vhi = ab[e, sl_hi] + bb[e, sl_hi] + chi
                ab[e, sl_lo] = jnp.maximum(vlo, 0.0)
                ab[e, sl_hi] = jnp.maximum(vhi, 0.0)
            return 0

        lax.fori_loop(0, C, edge, 0)

    def steady(k, b, bd):
        nb = 1 - b
        wait_idx(nb, (bd + 1) % 4)
        wait_sc(nb)
        issue_in(k + 1, nb, (bd + 1) % 4)
        wait_in(b)
        issue_idx(k + 2, b, (bd + 2) % 4)
        compute(b)
        issue_sc(b, bd)

    pltpu.sync_copy(zeros_hbm, agg_sh.at[pl.ds(row0, ROWS_PER_TILE)])
    plsc.subcore_barrier()

    pltpu.sync_copy(src3.at[wid, 0], is0)
    pltpu.sync_copy(dst3.at[wid, 0], id0)
    pltpu.sync_copy(src3.at[wid, 1], is1)
    pltpu.sync_copy(dst3.at[wid, 1], id1)
    issue_in(0, 0, 0)

    issue_in(1, 1, 1)
    wait_in(0)
    issue_idx(2, 0, 2)
    compute(0)
    issue_sc(0, 0)

    steady(1, 1, 1)
    steady(2, 0, 2)
    steady(3, 1, 3)

    def quad(t, _):
        for q in (0, 1, 2, 3):
            steady(4 + 4 * t + q, q % 2, q)
        return 0

    lax.fori_loop(0, (NCHUNK - 6) // 4, quad, 0)

    wait_idx(1, 1)
    wait_sc(1)
    issue_in(NCHUNK - 1, 1, 1)
    wait_in(0)
    compute(0)
    issue_sc(0, 0)

    wait_sc(0)
    wait_in(1)
    compute(1)
    issue_sc(1, 1)
    wait_sc(1)

    plsc.subcore_barrier()
    pltpu.sync_copy(agg_sh.at[pl.ds(row0, ROWS_PER_TILE)],
                    out_hbm.at[cc, pl.ds(row0, ROWS_PER_TILE)])


def _edge_phase(psrc, pdst, pe, src3, dst3, zeros_rows):
    mesh = plsc.VectorSubcoreMesh(core_axis_name="c", subcore_axis_name="s")
    return pl.kernel(
        _edge_body,
        out_type=jax.ShapeDtypeStruct((NC, NPAD, D), jnp.float32),
        mesh=mesh,
        compiler_params=pltpu.CompilerParams(needs_layout_passes=False),
        scratch_types=[
            pltpu.VMEM_SHARED((NPAD, D), jnp.float32),
            pltpu.VMEM((C,), jnp.int32),
            pltpu.VMEM((C,), jnp.int32),
            pltpu.VMEM((C,), jnp.int32),
            pltpu.VMEM((C,), jnp.int32),
            pltpu.VMEM((C,), jnp.int32),
            pltpu.VMEM((C,), jnp.int32),
            pltpu.VMEM((C, D), jnp.float32),
            pltpu.VMEM((C, D), jnp.float32),
            pltpu.VMEM((C, D // 2), jnp.int32),
            pltpu.VMEM((C, D), jnp.float32),
            pltpu.VMEM((C, D), jnp.float32),
            pltpu.VMEM((C, D // 2), jnp.int32),
            pltpu.SemaphoreType.DMA,
            pltpu.SemaphoreType.DMA,
            pltpu.SemaphoreType.DMA,
            pltpu.SemaphoreType.DMA,
        ],
    )(psrc, pdst, pe, src3, dst3, zeros_rows)


def kernel(x, edge_index, edge_attr, Wm0, bm0, Wu0, bu0, Wm1, bm1, Wu1, bu1):
    h0 = jnp.squeeze(x, -1)
    src3 = edge_index[0].reshape(NW, NCHUNK, C)
    dst3 = edge_index[1].reshape(NW, NCHUNK, C)
    zeros_rows = jnp.zeros((ROWS_PER_TILE, D), jnp.float32)

    pe0, pe1 = _tc_pe(edge_attr, Wm0[2 * D:], bm0, Wm1[2 * D:], bm1)

    psrc0, pdst0 = _tc_proj(h0, jnp.concatenate([Wm0[:D], Wm0[D:2 * D]], axis=1))
    agg0 = _edge_phase(psrc0, pdst0, pe0, src3, dst3, zeros_rows)
    h1, psrc1, pdst1 = _tc_update(
        h0, agg0, Wu0, bu0, act=True,
        extra_w=jnp.concatenate([Wm1[:D], Wm1[D:2 * D]], axis=1))

    agg1 = _edge_phase(psrc1, pdst1, pe1, src3, dst3, zeros_rows)
    (h2,) = _tc_update(h1, agg1, Wu1, bu1, act=False)

    return h2[:, :, None]


# speedup vs baseline: 1.1413x; 1.1413x over previous
import jax
import jax.numpy as jnp
import numpy as np
from jax import lax
from jax.experimental import pallas as pl
from jax.experimental.pallas import tpu as pltpu
from jax.experimental.pallas import tpu_sc as plsc

N = 10000
E = 320000
D = 128

NC = 2
NS = 16
NW = NC * NS
EPW = E // NW
C = 40
NCHUNK = EPW // C
NPAD = 10240
ROWS_PER_TILE = NPAD // NS


def _pe_body(x_ref, w_ref, b_ref, o0_ref, o1_ref):
    acc = jnp.dot(x_ref[...], w_ref[...], preferred_element_type=jnp.float32)
    acc = acc + b_ref[...]
    o0_ref[...] = acc[:, :D]
    o1_ref[...] = acc[:, D:]


def _tc_pe(edge_attr, we0, be0, we1, be1, block_r=2000):
    r, k = edge_attr.shape
    w = jnp.concatenate([we0, we1], axis=1)
    b = jnp.concatenate([be0, be1]).reshape(1, 2 * D)
    return pl.pallas_call(
        _pe_body,
        grid=(r // block_r,),
        in_specs=[
            pl.BlockSpec((block_r, k), lambda i: (i, 0)),
            pl.BlockSpec((k, 2 * D), lambda i: (0, 0)),
            pl.BlockSpec((1, 2 * D), lambda i: (0, 0)),
        ],
        out_specs=[
            pl.BlockSpec((block_r, D), lambda i: (i, 0)),
            pl.BlockSpec((block_r, D), lambda i: (i, 0)),
        ],
        out_shape=[
            jax.ShapeDtypeStruct((r, D), jnp.float32),
            jax.ShapeDtypeStruct((r, D), jnp.float32),
        ],
    )(edge_attr, w, b)


def _proj_body(h_ref, w_ref, o0_ref, o1_ref):
    acc = jnp.dot(h_ref[...], w_ref[...], preferred_element_type=jnp.float32)
    o0_ref[...] = acc[:, :D]
    o1_ref[...] = acc[:, D:]


def _tc_proj(h, wsrc_dst, block_r=400):
    r = h.shape[0]
    return pl.pallas_call(
        _proj_body,
        grid=(r // block_r,),
        in_specs=[
            pl.BlockSpec((block_r, D), lambda i: (i, 0)),
            pl.BlockSpec((D, 2 * D), lambda i: (0, 0)),
        ],
        out_specs=[
            pl.BlockSpec((block_r, D), lambda i: (i, 0)),
            pl.BlockSpec((block_r, D), lambda i: (i, 0)),
        ],
        out_shape=[
            jax.ShapeDtypeStruct((r, D), jnp.float32),
            jax.ShapeDtypeStruct((r, D), jnp.float32),
        ],
    )(h, wsrc_dst)


def _tc_update(h, agg, wu, bu, act, extra_w=None, block_r=400):
    r = h.shape[0]
    nout = 1 if extra_w is None else 3
    out_shape = [jax.ShapeDtypeStruct((r, D), jnp.float32)] * nout
    out_specs = [pl.BlockSpec((block_r, D), lambda i: (i, 0))] * nout

    def body(h_ref, agg_ref, wh_ref, wa_ref, b_ref, ew_ref, *o_refs):
        acc = jnp.dot(h_ref[...], wh_ref[...], preferred_element_type=jnp.float32)
        agg2 = agg_ref[0] + agg_ref[1]
        acc = acc + jnp.dot(agg2, wa_ref[...], preferred_element_type=jnp.float32)
        acc = acc + b_ref[...]
        if act:
            acc = jnp.maximum(acc, 0.0)
        o_refs[0][...] = acc
        if nout == 3:
            pr = jnp.dot(acc, ew_ref[...], preferred_element_type=jnp.float32)
            o_refs[1][...] = pr[:, :D]
            o_refs[2][...] = pr[:, D:]

    ew = extra_w if extra_w is not None else jnp.zeros((D, 2 * D), jnp.float32)
    return pl.pallas_call(
        body,
        grid=(r // block_r,),
        in_specs=[
            pl.BlockSpec((block_r, D), lambda i: (i, 0)),
            pl.BlockSpec((NC, block_r, D), lambda i: (0, i, 0)),
            pl.BlockSpec((D, D), lambda i: (0, 0)),
            pl.BlockSpec((D, D), lambda i: (0, 0)),
            pl.BlockSpec((1, D), lambda i: (0, 0)),
            pl.BlockSpec((D, 2 * D), lambda i: (0, 0)),
        ],
        out_specs=out_specs,
        out_shape=out_shape,
    )(h, agg, wu[:D], wu[D:], bu.reshape(1, D), ew)


def _edge_body(psrc, pdst, pe, src3, dst3, zeros_hbm, out_hbm,
               agg_sh, is0, is1, id0, id1, id2, id3,
               a0, b0, c0, a1, b1, c1,
               sem_i, sem_in0, sem_in1, sem_sc):
    cc = lax.axis_index("c")
    ss = lax.axis_index("s")
    wid = cc * NS + ss
    row0 = ss * ROWS_PER_TILE

    IS = (is0, is1)
    ID = (id0, id1, id2, id3)
    A = (a0, a1)
    B = (b0, b1)
    CB = (c0, c1)
    SIN = (sem_in0, sem_in1)

    def issue_idx(k, bs, bd):
        pltpu.async_copy(src3.at[wid, k], IS[bs], sem_i)
        pltpu.async_copy(dst3.at[wid, k], ID[bd], sem_i)

    def wait_idx(bs, bd):
        pltpu.make_async_copy(src3.at[0, 0], IS[bs], sem_i).wait()
        pltpu.make_async_copy(dst3.at[0, 0], ID[bd], sem_i).wait()

    def issue_in(k, b, bd):
        base = wid * EPW + k * C
        pltpu.async_copy(psrc.at[IS[b]], A[b], SIN[b])
        pltpu.async_copy(pdst.at[ID[bd]], B[b], SIN[b])
        pltpu.async_copy(pe.at[pl.ds(base, C)], CB[b], SIN[b])

    def wait_in(b):
        pltpu.make_async_copy(zeros_hbm.at[pl.ds(0, C)], A[b], SIN[b]).wait()
        pltpu.make_async_copy(zeros_hbm.at[pl.ds(0, C)], B[b], SIN[b]).wait()
        pltpu.make_async_copy(pe.at[pl.ds(0, C)], CB[b], SIN[b]).wait()

    def issue_sc(b, bd):
        pltpu.async_copy(A[b], agg_sh.at[ID[bd]], sem_sc, add=True)

    def wait_sc(b):
        pltpu.make_async_copy(zeros_hbm.at[pl.ds(0, C)], A[b], sem_sc).wait()

    def compute(b):
        ab, bb, cb = A[b], B[b], CB[b]

        def edge(e2, _):
            for u in range(2):
                e = 2 * e2 + u
                for j in range(D // 16):
                    sl = pl.ds(j * 16, 16)
                    v = ab[e, sl] + bb[e, sl] + cb[e, sl]
                    ab[e, sl] = jnp.maximum(v, 0.0)
            return 0

        lax.fori_loop(0, C // 2, edge, 0)

    def steady(k, b, bd):
        nb = 1 - b
        wait_idx(nb, (bd + 1) % 4)
        wait_sc(nb)
        issue_in(k + 1, nb, (bd + 1) % 4)
        wait_in(b)
        issue_idx(k + 2, b, (bd + 2) % 4)
        compute(b)
        issue_sc(b, bd)

    pltpu.sync_copy(zeros_hbm, agg_sh.at[pl.ds(row0, ROWS_PER_TILE)])
    plsc.subcore_barrier()

    pltpu.sync_copy(src3.at[wid, 0], is0)
    pltpu.sync_copy(dst3.at[wid, 0], id0)
    pltpu.sync_copy(src3.at[wid, 1], is1)
    pltpu.sync_copy(dst3.at[wid, 1], id1)
    issue_in(0, 0, 0)

    issue_in(1, 1, 1)
    wait_in(0)
    issue_idx(2, 0, 2)
    compute(0)
    issue_sc(0, 0)

    steady(1, 1, 1)
    steady(2, 0, 2)
    steady(3, 1, 3)

    def quad(t, _):
        for q in (0, 1, 2, 3):
            steady(4 + 4 * t + q, q % 2, q)
        return 0

    lax.fori_loop(0, (NCHUNK - 6) // 4, quad, 0)

    wait_idx(1, 1)
    wait_sc(1)
    issue_in(NCHUNK - 1, 1, 1)
    wait_in(0)
    compute(0)
    issue_sc(0, 0)

    wait_sc(0)
    wait_in(1)
    compute(1)
    issue_sc(1, 1)
    wait_sc(1)

    plsc.subcore_barrier()
    pltpu.sync_copy(agg_sh.at[pl.ds(row0, ROWS_PER_TILE)],
                    out_hbm.at[cc, pl.ds(row0, ROWS_PER_TILE)])


def _edge_phase(psrc, pdst, pe, src3, dst3, zeros_rows):
    mesh = plsc.VectorSubcoreMesh(core_axis_name="c", subcore_axis_name="s")
    return pl.kernel(
        _edge_body,
        out_type=jax.ShapeDtypeStruct((NC, NPAD, D), jnp.float32),
        mesh=mesh,
        scratch_types=[
            pltpu.VMEM_SHARED((NPAD, D), jnp.float32),
            pltpu.VMEM((C,), jnp.int32),
            pltpu.VMEM((C,), jnp.int32),
            pltpu.VMEM((C,), jnp.int32),
            pltpu.VMEM((C,), jnp.int32),
            pltpu.VMEM((C,), jnp.int32),
            pltpu.VMEM((C,), jnp.int32),
            pltpu.VMEM((C, D), jnp.float32),
            pltpu.VMEM((C, D), jnp.float32),
            pltpu.VMEM((C, D), jnp.float32),
            pltpu.VMEM((C, D), jnp.float32),
            pltpu.VMEM((C, D), jnp.float32),
            pltpu.VMEM((C, D), jnp.float32),
            pltpu.SemaphoreType.DMA,
            pltpu.SemaphoreType.DMA,
            pltpu.SemaphoreType.DMA,
            pltpu.SemaphoreType.DMA,
        ],
    )(psrc, pdst, pe, src3, dst3, zeros_rows)


def kernel(x, edge_index, edge_attr, Wm0, bm0, Wu0, bu0, Wm1, bm1, Wu1, bu1):
    h0 = jnp.squeeze(x, -1)
    src3 = edge_index[0].reshape(NW, NCHUNK, C)
    dst3 = edge_index[1].reshape(NW, NCHUNK, C)
    zeros_rows = jnp.zeros((ROWS_PER_TILE, D), jnp.float32)

    pe0, pe1 = _tc_pe(edge_attr, Wm0[2 * D:], bm0, Wm1[2 * D:], bm1)

    psrc0, pdst0 = _tc_proj(h0, jnp.concatenate([Wm0[:D], Wm0[D:2 * D]], axis=1))
    agg0 = _edge_phase(psrc0, pdst0, pe0, src3, dst3, zeros_rows)
    h1, psrc1, pdst1 = _tc_update(
        h0, agg0, Wu0, bu0, act=True,
        extra_w=jnp.concatenate([Wm1[:D], Wm1[D:2 * D]], axis=1))

    agg1 = _edge_phase(psrc1, pdst1, pe1, src3, dst3, zeros_rows)
    (h2,) = _tc_update(h1, agg1, Wu1, bu1, act=False)

    return h2[:, :, None]
